# Initial kernel scaffold; baseline (speedup 1.0000x reference)
import jax
import jax.numpy as jnp
from jax.experimental import pallas as pl


def kernel(x_tx, x_user, ei_pays, ei_own, Wl_pays, Wr_pays, b_pays, Wl_own, Wr_own, b_own, W_out, b_out):
    raise NotImplementedError("write your pallas kernel here")


# speedup vs baseline: 6.6126x; 6.6126x over previous
import functools

import jax
import jax.numpy as jnp
from jax import lax
from jax.experimental import pallas as pl
from jax.experimental.pallas import tpu as pltpu
from jax.experimental.pallas import tpu_sc as plsc

N_TX = 10000
N_USER = 10000
D = 128
H = 64
E = 320000

W = 80
NC = 2
NS = 16
NW = NC * NS
EPW = E // NW
CH = 80
NCHUNK = EPW // CH
CPR = 1000
NCW = N_TX // CPR
ZR = 200
BR = 1000


def _pre_body(xu_ref, wl_ref, out_ref):
    mm = jnp.dot(xu_ref[...], wl_ref[...], preferred_element_type=jnp.float32)
    col = lax.broadcasted_iota(jnp.int32, (BR, W - H), 1)
    extra = jnp.where(col == 0, jnp.float32(1.0), jnp.float32(0.0))
    out_ref[...] = jnp.concatenate([mm, extra], axis=1)


def _pre(x_user, Wl):
    return pl.pallas_call(
        _pre_body,
        grid=(N_USER // BR,),
        in_specs=[
            pl.BlockSpec((BR, D), lambda i: (i, 0)),
            pl.BlockSpec((D, H), lambda i: (0, 0)),
        ],
        out_specs=pl.BlockSpec((BR, W), lambda i: (i, 0)),
        out_shape=jax.ShapeDtypeStruct((N_USER, W), jnp.float32),
    )(x_user, Wl)


def _seg_body(yaug_hbm, src_hbm, dst_hbm, out_hbm,
              src_v, dst_v, rows_v, zrow_v, acc_sh, sem):
    cid = lax.axis_index("c")
    sid = lax.axis_index("s")
    wid = sid * NC + cid

    @pl.when(sid < NCW)
    def _():
        def zfill(r, c):
            for j in range(W // 16):
                zrow_v[r, pl.ds(j * 16, 16)] = jnp.zeros((16,), jnp.float32)
            return c
        lax.fori_loop(0, ZR, zfill, 0)

        def zcopy(t, c):
            pltpu.sync_copy(zrow_v, acc_sh.at[pl.ds(sid * CPR + t * ZR, ZR)])
            return c
        lax.fori_loop(0, CPR // ZR, zcopy, 0)
    plsc.subcore_barrier()

    ebase = wid * EPW

    def body(i, c):
        off = pl.multiple_of(ebase + i * CH, 8)
        pltpu.sync_copy(src_hbm.at[pl.ds(off, CH)], src_v)
        pltpu.sync_copy(dst_hbm.at[pl.ds(off, CH)], dst_v)
        pltpu.async_copy(yaug_hbm.at[src_v], rows_v, sem).wait()
        pltpu.sync_copy(rows_v, acc_sh.at[dst_v], add=True)
        return c
    lax.fori_loop(0, NCHUNK, body, 0)
    plsc.subcore_barrier()

    @pl.when(sid < NCW)
    def _():
        pltpu.sync_copy(acc_sh.at[pl.ds(sid * CPR, CPR)],
                        out_hbm.at[cid, pl.ds(sid * CPR, CPR)])


def _seg(yaug, src, dst):
    mesh = plsc.VectorSubcoreMesh(core_axis_name="c", subcore_axis_name="s")
    f = functools.partial(
        pl.kernel,
        mesh=mesh,
        compiler_params=pltpu.CompilerParams(use_tc_tiling_on_sc=False),
        out_type=jax.ShapeDtypeStruct((NC, N_TX, W), jnp.float32),
        scratch_types=[
            pltpu.VMEM((CH,), jnp.int32),
            pltpu.VMEM((CH,), jnp.int32),
            pltpu.VMEM((CH, W), jnp.float32),
            pltpu.VMEM((ZR, W), jnp.float32),
            pltpu.VMEM_SHARED((N_TX, W), jnp.float32),
            pltpu.SemaphoreType.DMA,
        ],
    )(_seg_body)
    return f(yaug, src, dst)


def _post_body(p_ref, x_ref, wr_ref, b_ref, wo_ref, bo_ref, h_ref, lg_ref):
    acc = p_ref[0] + p_ref[1]
    mean = acc[:, :H] / jnp.maximum(acc[:, H:H + 1], 1.0)
    z = jnp.dot(x_ref[...], wr_ref[...], preferred_element_type=jnp.float32)
    h = jnp.maximum(mean + z + b_ref[...], 0.0)
    h_ref[...] = h
    lg_ref[...] = jnp.sum(h * wo_ref[...], axis=1, keepdims=True) + bo_ref[...]


def _post(parts, x_tx, Wr, b, wo, bo):
    return pl.pallas_call(
        _post_body,
        grid=(N_TX // BR,),
        in_specs=[
            pl.BlockSpec((NC, BR, W), lambda i: (0, i, 0)),
            pl.BlockSpec((BR, D), lambda i: (i, 0)),
            pl.BlockSpec((D, H), lambda i: (0, 0)),
            pl.BlockSpec((1, H), lambda i: (0, 0)),
            pl.BlockSpec((1, H), lambda i: (0, 0)),
            pl.BlockSpec((1, 1), lambda i: (0, 0)),
        ],
        out_specs=[
            pl.BlockSpec((BR, H), lambda i: (i, 0)),
            pl.BlockSpec((BR, 1), lambda i: (i, 0)),
        ],
        out_shape=[
            jax.ShapeDtypeStruct((N_TX, H), jnp.float32),
            jax.ShapeDtypeStruct((N_TX, 1), jnp.float32),
        ],
    )(parts, x_tx, Wr, b, wo, bo)


def kernel(x_tx, x_user, ei_pays, ei_own, Wl_pays, Wr_pays, b_pays,
           Wl_own, Wr_own, b_own, W_out, b_out):
    yaug = _pre(x_user, Wl_pays)
    parts = _seg(yaug, ei_pays[0], ei_pays[1])
    h, lg = _post(parts, x_tx, Wr_pays,
                  b_pays.reshape(1, H), W_out.reshape(1, H),
                  b_out.reshape(1, 1))
    return (lg.reshape(N_TX), h)


# speedup vs baseline: 15.9381x; 2.4102x over previous
import functools

import jax
import jax.numpy as jnp
from jax import lax
from jax.experimental import pallas as pl
from jax.experimental.pallas import tpu as pltpu
from jax.experimental.pallas import tpu_sc as plsc

N_TX = 10000
N_USER = 10000
D = 128
H = 64
E = 320000

W = 80
NC = 2
NS = 16
NW = NC * NS
EPW = E // NW
CH = 80
NCHUNK = EPW // CH
NB = 4
CPR = 1000
NCW = N_TX // CPR
ZR = 200
BR = 1000


def _pre_body(xu_ref, wl_ref, out_ref):
    mm = jnp.dot(xu_ref[...], wl_ref[...], preferred_element_type=jnp.float32)
    col = lax.broadcasted_iota(jnp.int32, (BR, W - H), 1)
    extra = jnp.where(col == 0, jnp.float32(1.0), jnp.float32(0.0))
    out_ref[...] = jnp.concatenate([mm, extra], axis=1)


def _pre(x_user, Wl):
    return pl.pallas_call(
        _pre_body,
        grid=(N_USER // BR,),
        in_specs=[
            pl.BlockSpec((BR, D), lambda i: (i, 0)),
            pl.BlockSpec((D, H), lambda i: (0, 0)),
        ],
        out_specs=pl.BlockSpec((BR, W), lambda i: (i, 0)),
        out_shape=jax.ShapeDtypeStruct((N_USER, W), jnp.float32),
    )(x_user, Wl)


def _seg_body(yaug_hbm, src_hbm, dst_hbm, out_hbm,
              src_v, dst_v, rows_v, zrow_v, acc_sh, gsem):
    cid = lax.axis_index("c")
    sid = lax.axis_index("s")
    wid = sid * NC + cid

    pltpu.sync_copy(src_hbm.at[wid], src_v)
    pltpu.sync_copy(dst_hbm.at[wid], dst_v)

    @pl.when(sid < NCW)
    def _():
        def zfill(r, c):
            for j in range(W // 16):
                zrow_v[r, pl.ds(j * 16, 16)] = jnp.zeros((16,), jnp.float32)
            return c
        lax.fori_loop(0, ZR, zfill, 0)

        def zcopy(t, c):
            pltpu.sync_copy(zrow_v, acc_sh.at[pl.ds(sid * CPR + t * ZR, ZR)])
            return c
        lax.fori_loop(0, CPR // ZR, zcopy, 0)
    plsc.subcore_barrier()

    def fire(j, s):
        pltpu.async_copy(yaug_hbm.at[src_v.at[j]], rows_v.at[s], gsem.at[s])

    for s in range(NB):
        fire(s, s)

    def body(j, c):
        s = lax.rem(j, NB)
        pltpu.make_async_copy(yaug_hbm.at[src_v.at[j]], rows_v.at[s],
                              gsem.at[s]).wait()
        pltpu.sync_copy(rows_v.at[s], acc_sh.at[dst_v.at[j]], add=True)
        jn = j + NB

        @pl.when(jn < NCHUNK)
        def _():
            fire(jn, s)
        return c
    lax.fori_loop(0, NCHUNK, body, 0)
    plsc.subcore_barrier()

    @pl.when(sid < NCW)
    def _():
        pltpu.sync_copy(acc_sh.at[pl.ds(sid * CPR, CPR)],
                        out_hbm.at[cid, pl.ds(sid * CPR, CPR)])


def _seg(yaug, src, dst):
    mesh = plsc.VectorSubcoreMesh(core_axis_name="c", subcore_axis_name="s")
    f = functools.partial(
        pl.kernel,
        mesh=mesh,
        compiler_params=pltpu.CompilerParams(use_tc_tiling_on_sc=False),
        out_type=jax.ShapeDtypeStruct((NC, N_TX, W), jnp.float32),
        scratch_types=[
            pltpu.VMEM((NCHUNK, CH), jnp.int32),
            pltpu.VMEM((NCHUNK, CH), jnp.int32),
            pltpu.VMEM((NB, CH, W), jnp.float32),
            pltpu.VMEM((ZR, W), jnp.float32),
            pltpu.VMEM_SHARED((N_TX, W), jnp.float32),
            pltpu.SemaphoreType.DMA((NB,)),
        ],
    )(_seg_body)
    return f(yaug, src.reshape(NW, NCHUNK, CH), dst.reshape(NW, NCHUNK, CH))


def _post_body(p_ref, x_ref, wr_ref, b_ref, wo_ref, bo_ref, h_ref, lg_ref):
    acc = p_ref[0] + p_ref[1]
    mean = acc[:, :H] / jnp.maximum(acc[:, H:H + 1], 1.0)
    z = jnp.dot(x_ref[...], wr_ref[...], preferred_element_type=jnp.float32)
    h = jnp.maximum(mean + z + b_ref[...], 0.0)
    h_ref[...] = h
    lg_ref[...] = jnp.sum(h * wo_ref[...], axis=1, keepdims=True) + bo_ref[...]


def _post(parts, x_tx, Wr, b, wo, bo):
    return pl.pallas_call(
        _post_body,
        grid=(N_TX // BR,),
        in_specs=[
            pl.BlockSpec((NC, BR, W), lambda i: (0, i, 0)),
            pl.BlockSpec((BR, D), lambda i: (i, 0)),
            pl.BlockSpec((D, H), lambda i: (0, 0)),
            pl.BlockSpec((1, H), lambda i: (0, 0)),
            pl.BlockSpec((1, H), lambda i: (0, 0)),
            pl.BlockSpec((1, 1), lambda i: (0, 0)),
        ],
        out_specs=[
            pl.BlockSpec((BR, H), lambda i: (i, 0)),
            pl.BlockSpec((BR, 1), lambda i: (i, 0)),
        ],
        out_shape=[
            jax.ShapeDtypeStruct((N_TX, H), jnp.float32),
            jax.ShapeDtypeStruct((N_TX, 1), jnp.float32),
        ],
    )(parts, x_tx, Wr, b, wo, bo)


def kernel(x_tx, x_user, ei_pays, ei_own, Wl_pays, Wr_pays, b_pays,
           Wl_own, Wr_own, b_own, W_out, b_out):
    yaug = _pre(x_user, Wl_pays)
    parts = _seg(yaug, ei_pays[0], ei_pays[1])
    h, lg = _post(parts, x_tx, Wr_pays,
                  b_pays.reshape(1, H), W_out.reshape(1, H),
                  b_out.reshape(1, 1))
    return (lg.reshape(N_TX), h)
